# baseline (speedup 1.0000x reference)
import functools

import jax
import jax.numpy as jnp
from jax import lax
from jax.experimental import pallas as pl
from jax.experimental.pallas import tpu as pltpu
from jax.experimental.pallas import tpu_sc as plsc

N_NODES = 50000
DIM = 128
D_FEAT = 256
D_OUT = DIM + D_FEAT

CHUNK = 128
NUM_FULL = N_NODES // CHUNK
NUM_CHUNKS = NUM_FULL + (1 if N_NODES % CHUNK else 0)
TAIL_BASE = N_NODES - CHUNK


@functools.lru_cache(maxsize=None)
def _build():
    mesh = plsc.VectorSubcoreMesh(core_axis_name="c", subcore_axis_name="s")
    nc, ns = mesh.num_cores, mesh.num_subcores
    nw = nc * ns
    iters = -(-NUM_CHUNKS // nw)

    @functools.partial(
        pl.kernel,
        out_type=jax.ShapeDtypeStruct((N_NODES, D_OUT), jnp.float32),
        mesh=mesh,
        scratch_types=[
            pltpu.VMEM((iters, CHUNK), jnp.int32),
            pltpu.VMEM((2, CHUNK, DIM), jnp.float32),
            pltpu.SemaphoreType.DMA,
            pltpu.SemaphoreType.DMA,
            pltpu.SemaphoreType.DMA,
            pltpu.SemaphoreType.DMA,
            pltpu.SemaphoreType.DMA,
            pltpu.SemaphoreType.DMA,
        ],
    )
    def body(x_hbm, lt_hbm, idx_hbm, out_hbm, idx_v, h_v, isem,
             gsem0, gsem1, wsem0, wsem1, xsem):
        gsem = (gsem0, gsem1)
        wsem = (wsem0, wsem1)
        wid = lax.axis_index("s") * nc + lax.axis_index("c")

        def chunk_base(i):
            c = jnp.minimum(wid + i * nw, NUM_CHUNKS - 1)
            b = jnp.where(c < NUM_FULL, c * CHUNK, TAIL_BASE)
            return pl.multiple_of(b, 8)

        bases = [chunk_base(i) for i in range(iters)]

        idx_cp = [
            pltpu.async_copy(idx_hbm.at[pl.ds(bases[i], CHUNK)], idx_v.at[i], isem)
            for i in range(iters)
        ]
        for cp in idx_cp:
            cp.wait()

        gathers = [None, None]
        writes = [None, None]
        x_cp = []
        for i in range(iters):
            p = i % 2
            if writes[p] is not None:
                writes[p].wait()
            gathers[p] = pltpu.async_copy(lt_hbm.at[idx_v.at[i]], h_v.at[p], gsem[p])
            x_cp.append(pltpu.async_copy(
                x_hbm.at[pl.ds(bases[i], CHUNK)],
                out_hbm.at[pl.ds(bases[i], CHUNK), pl.ds(DIM, D_FEAT)],
                xsem))
            if i >= 1:
                q = (i - 1) % 2
                gathers[q].wait()
                writes[q] = pltpu.async_copy(
                    h_v.at[q],
                    out_hbm.at[pl.ds(bases[i - 1], CHUNK), pl.ds(0, DIM)],
                    wsem[q])
        p = (iters - 1) % 2
        gathers[p].wait()
        writes[p] = pltpu.async_copy(
            h_v.at[p],
            out_hbm.at[pl.ds(bases[iters - 1], CHUNK), pl.ds(0, DIM)],
            wsem[p])
        for w in writes:
            if w is not None:
                w.wait()
        for cp in x_cp:
            cp.wait()

    return body


def kernel(x, lt, all_nodes):
    idx32 = all_nodes.astype(jnp.int32)
    return _build()(x, lt, idx32)


# speedup vs baseline: 20.2904x; 20.2904x over previous
import functools

import jax
import jax.numpy as jnp
from jax import lax
from jax.experimental import pallas as pl
from jax.experimental.pallas import tpu as pltpu
from jax.experimental.pallas import tpu_sc as plsc

N_NODES = 50000
DIM = 128
D_FEAT = 256
D_OUT = DIM + D_FEAT

CHUNK = 128
NUM_FULL = N_NODES // CHUNK
NUM_CHUNKS = NUM_FULL + (1 if N_NODES % CHUNK else 0)
TAIL_BASE = N_NODES - CHUNK


@functools.lru_cache(maxsize=None)
def _build():
    mesh = plsc.VectorSubcoreMesh(core_axis_name="c", subcore_axis_name="s")
    nc, ns = mesh.num_cores, mesh.num_subcores
    nw = nc * ns
    iters = -(-NUM_CHUNKS // nw)

    @functools.partial(
        pl.kernel,
        out_type=jax.ShapeDtypeStruct((N_NODES, D_OUT), jnp.float32),
        mesh=mesh,
        scratch_types=[
            pltpu.VMEM((iters, CHUNK), jnp.int32),
            pltpu.VMEM((2, CHUNK, DIM), jnp.float32),
            pltpu.VMEM((2, CHUNK, D_FEAT), jnp.float32),
        ] + [pltpu.SemaphoreType.DMA] * 9,
    )
    def body(x_hbm, lt_hbm, idx_hbm, out_hbm, idx_v, h_v, x_v, isem,
             gsem0, gsem1, wsem0, wsem1, xrsem0, xrsem1, xwsem0, xwsem1):
        gsem = (gsem0, gsem1)
        wsem = (wsem0, wsem1)
        xrsem = (xrsem0, xrsem1)
        xwsem = (xwsem0, xwsem1)
        wid = lax.axis_index("s") * nc + lax.axis_index("c")

        def chunk_base(i):
            c = jnp.minimum(wid + i * nw, NUM_CHUNKS - 1)
            b = jnp.where(c < NUM_FULL, c * CHUNK, TAIL_BASE)
            return pl.multiple_of(b, 8)

        bases = [chunk_base(i) for i in range(iters)]

        idx_cp = [
            pltpu.async_copy(idx_hbm.at[pl.ds(bases[i], CHUNK)], idx_v.at[i], isem)
            for i in range(iters)
        ]
        for cp in idx_cp:
            cp.wait()

        gathers = [None, None]
        writes = [None, None]
        xreads = [None, None]
        xwrites = [None, None]
        for i in range(iters):
            p = i % 2
            if writes[p] is not None:
                writes[p].wait()
            if xwrites[p] is not None:
                xwrites[p].wait()
            gathers[p] = pltpu.async_copy(lt_hbm.at[idx_v.at[i]], h_v.at[p], gsem[p])
            xreads[p] = pltpu.async_copy(
                x_hbm.at[pl.ds(bases[i], CHUNK)], x_v.at[p], xrsem[p])
            if i >= 1:
                q = (i - 1) % 2
                gathers[q].wait()
                writes[q] = pltpu.async_copy(
                    h_v.at[q],
                    out_hbm.at[pl.ds(bases[i - 1], CHUNK), pl.ds(0, DIM)],
                    wsem[q])
                xreads[q].wait()
                xwrites[q] = pltpu.async_copy(
                    x_v.at[q],
                    out_hbm.at[pl.ds(bases[i - 1], CHUNK), pl.ds(DIM, D_FEAT)],
                    xwsem[q])
        p = (iters - 1) % 2
        gathers[p].wait()
        writes[p] = pltpu.async_copy(
            h_v.at[p],
            out_hbm.at[pl.ds(bases[iters - 1], CHUNK), pl.ds(0, DIM)],
            wsem[p])
        xreads[p].wait()
        xwrites[p] = pltpu.async_copy(
            x_v.at[p],
            out_hbm.at[pl.ds(bases[iters - 1], CHUNK), pl.ds(DIM, D_FEAT)],
            xwsem[p])
        for cp in writes + xwrites:
            if cp is not None:
                cp.wait()

    return body


def kernel(x, lt, all_nodes):
    idx32 = all_nodes.astype(jnp.int32)
    return _build()(x, lt, idx32)
